# Initial kernel scaffold; baseline (speedup 1.0000x reference)
import jax
import jax.numpy as jnp
from jax.experimental import pallas as pl


def kernel(x, edge_index, W, b, gamma, beta):
    raise NotImplementedError("write your pallas kernel here")


# speedup vs baseline: 10.7916x; 10.7916x over previous
import functools

import jax
import jax.numpy as jnp
from jax import lax
from jax.experimental import pallas as pl
from jax.experimental.pallas import tpu as pltpu
from jax.experimental.pallas import tpu_sc as plsc

NC = 2
NS = 16
NW = NC * NS
LANE = 128


def _deg_kernel(nb, n_acc, rpt):
    mesh = plsc.VectorSubcoreMesh(core_axis_name="c", subcore_axis_name="s")

    @functools.partial(
        pl.kernel,
        mesh=mesh,
        out_type=jax.ShapeDtypeStruct((NC * n_acc,), jnp.float32),
        scratch_types=[
            pltpu.VMEM_SHARED((n_acc,), jnp.float32),
            pltpu.VMEM((nb, LANE), jnp.int32),
            pltpu.VMEM((LANE,), jnp.float32),
            pltpu.VMEM((rpt,), jnp.float32),
        ],
    )
    def deg_k(dst_hbm, zeros_hbm, ones_hbm, out_hbm, acc, dstv, ones, zbuf):
        c = lax.axis_index("c")
        s = lax.axis_index("s")
        wid = c * NS + s
        pltpu.sync_copy(zeros_hbm, zbuf)
        pltpu.sync_copy(zbuf, acc.at[pl.ds(s * rpt, rpt)])
        pltpu.sync_copy(ones_hbm, ones)
        pltpu.sync_copy(dst_hbm.at[pl.ds(wid * nb, nb)], dstv)
        plsc.subcore_barrier()

        @pl.loop(0, nb)
        def _(j):
            pltpu.sync_copy(ones, acc.at[dstv.at[j]], add=True)

        plsc.subcore_barrier()
        pltpu.sync_copy(acc.at[pl.ds(s * rpt, rpt)], zbuf)
        pltpu.sync_copy(zbuf, out_hbm.at[pl.ds(c * n_acc + s * rpt, rpt)])

    return deg_k


def _scatter_kernel(nb, n_acc, rpt, d):
    mesh = plsc.VectorSubcoreMesh(core_axis_name="c", subcore_axis_name="s")

    @functools.partial(
        pl.kernel,
        mesh=mesh,
        out_type=jax.ShapeDtypeStruct((NC, n_acc, d), jnp.float32),
        scratch_types=[
            pltpu.VMEM_SHARED((n_acc, d), jnp.float32),
            pltpu.VMEM((nb, LANE), jnp.int32),
            pltpu.VMEM((nb, LANE), jnp.int32),
            pltpu.VMEM((LANE, d), jnp.float32),
            pltpu.SemaphoreType.DMA,
        ],
    )
    def scat_k(y_hbm, src_hbm, dst_hbm, zeros_hbm, out_hbm,
               acc, srcv, dstv, buf, sem):
        c = lax.axis_index("c")
        s = lax.axis_index("s")
        wid = c * NS + s
        pltpu.sync_copy(zeros_hbm, buf)
        for k in range(rpt // LANE):
            pltpu.sync_copy(buf, acc.at[pl.ds(s * rpt + k * LANE, LANE)])
        pltpu.sync_copy(src_hbm.at[pl.ds(wid * nb, nb)], srcv)
        pltpu.sync_copy(dst_hbm.at[pl.ds(wid * nb, nb)], dstv)
        plsc.subcore_barrier()

        @pl.loop(0, nb)
        def _(j):
            pltpu.async_copy(y_hbm.at[srcv.at[j]], buf, sem).wait()
            pltpu.sync_copy(buf, acc.at[dstv.at[j]], add=True)

        plsc.subcore_barrier()
        for k in range(rpt // LANE):
            pltpu.sync_copy(acc.at[pl.ds(s * rpt + k * LANE, LANE)], buf)
            pltpu.sync_copy(buf, out_hbm.at[c, pl.ds(s * rpt + k * LANE, LANE)])

    return scat_k


def _mm_body(x_ref, w_ref, deg_ref, y_ref, dinv_ref):
    xw = jnp.dot(x_ref[...], w_ref[...], preferred_element_type=jnp.float32)
    deg = deg_ref[0] + deg_ref[1] + 1.0
    dinv = lax.rsqrt(deg)
    y_ref[...] = xw * dinv
    dinv_ref[...] = dinv


def _final_body(p_ref, y_ref, dinv_ref, x_ref, b_ref, g_ref, be_ref, o_ref):
    agg = (p_ref[0] + p_ref[1] + y_ref[...]) * dinv_ref[...] + b_ref[...]
    mu = jnp.mean(agg, axis=-1, keepdims=True)
    var = jnp.mean((agg - mu) ** 2, axis=-1, keepdims=True)
    h = (agg - mu) * lax.rsqrt(var + 1e-5) * g_ref[...] + be_ref[...]
    h = jnp.where(h >= 0, h, 0.01 * h)
    o_ref[...] = h + x_ref[...]


def kernel(x, edge_index, W, b, gamma, beta):
    n, d = x.shape
    e = edge_index.shape[1]
    nb = (-(-e // (NW * LANE)) + 7) // 8 * 8
    e_pad = NW * nb * LANE
    rpt = (-(-n // NS) + LANE - 1) // LANE * LANE
    n_acc = rpt * NS

    src = edge_index[0]
    dst = edge_index[1]
    pad = e_pad - e
    src_p = jnp.concatenate(
        [src, jnp.zeros((pad,), jnp.int32)]).reshape(NW * nb, LANE)
    dst_p = jnp.concatenate(
        [dst, jnp.full((pad,), n_acc - 1, jnp.int32)]).reshape(NW * nb, LANE)

    zeros1 = jnp.zeros((rpt,), jnp.float32)
    ones1 = jnp.ones((LANE,), jnp.float32)
    zeros2 = jnp.zeros((LANE, d), jnp.float32)

    degp = _deg_kernel(nb, n_acc, rpt)(dst_p, zeros1, ones1)
    deg3 = degp.reshape(NC, n_acc, 1)

    B = 512
    G = -(-n // B)
    y, dinv = pl.pallas_call(
        _mm_body,
        grid=(G,),
        in_specs=[
            pl.BlockSpec((B, d), lambda i: (i, 0)),
            pl.BlockSpec((d, d), lambda i: (0, 0)),
            pl.BlockSpec((NC, B, 1), lambda i: (0, i, 0)),
        ],
        out_specs=[
            pl.BlockSpec((B, d), lambda i: (i, 0)),
            pl.BlockSpec((B, 1), lambda i: (i, 0)),
        ],
        out_shape=[
            jax.ShapeDtypeStruct((n, d), jnp.float32),
            jax.ShapeDtypeStruct((n, 1), jnp.float32),
        ],
    )(x, W, deg3)

    partials = _scatter_kernel(nb, n_acc, rpt, d)(y, src_p, dst_p, zeros2)
    part = partials[:, :n, :]

    out = pl.pallas_call(
        _final_body,
        grid=(G,),
        in_specs=[
            pl.BlockSpec((NC, B, d), lambda i: (0, i, 0)),
            pl.BlockSpec((B, d), lambda i: (i, 0)),
            pl.BlockSpec((B, 1), lambda i: (i, 0)),
            pl.BlockSpec((B, d), lambda i: (i, 0)),
            pl.BlockSpec((1, d), lambda i: (0, 0)),
            pl.BlockSpec((1, d), lambda i: (0, 0)),
            pl.BlockSpec((1, d), lambda i: (0, 0)),
        ],
        out_specs=pl.BlockSpec((B, d), lambda i: (i, 0)),
        out_shape=jax.ShapeDtypeStruct((n, d), jnp.float32),
    )(part, y, dinv, x,
      b.reshape(1, d), gamma.reshape(1, d), beta.reshape(1, d))
    return out
